# baseline (speedup 1.0000x reference)
import functools

import jax
import jax.numpy as jnp
from jax import lax
from jax.experimental import pallas as pl
from jax.experimental.pallas import tpu as pltpu
from jax.experimental.pallas import tpu_sc as plsc

E = 8
NE = 10
EP = 16
S = 2048
H = 1024
FF = 4096
T = 512
G = 16
FC = 1024
KFN = FF // FC
NROWS = G * T
TRASH = NROWS - 1
NC, NS = 2, 16
NW = NC * NS
TPW = S // NW
CCH = 16


def _router_body(x_ref, wg_ref, logits_ref, dst0_ref, dst1_ref,
                 wb0_ref, wb1_ref, cnts_ref):
    x = x_ref[...]
    wg = wg_ref[...]
    logits = lax.dot_general(x, wg, (((1,), (1,)), ((), ())),
                             preferred_element_type=jnp.float32)
    col = lax.broadcasted_iota(jnp.int32, (S, EP), 1)
    lm = jnp.where(col < NE, logits, jnp.float32(-1e30))
    m = jnp.max(lm, axis=1, keepdims=True)
    p = jnp.exp(lm - m)
    probs = p / jnp.sum(p, axis=1, keepdims=True)
    v1 = jnp.max(probs, axis=1, keepdims=True)
    i1 = jnp.min(jnp.where(probs == v1, col, EP + 1), axis=1, keepdims=True)
    probs2 = jnp.where(col == i1, jnp.float32(-1.0), probs)
    v2 = jnp.max(probs2, axis=1, keepdims=True)
    i2 = jnp.min(jnp.where(probs2 == v2, col, EP + 1), axis=1, keepdims=True)
    real1 = (i1 < E).astype(jnp.float32)
    real2 = (i2 < E).astype(jnp.float32)
    ssum = v1 * real1 + v2 * real2
    denom = jnp.where(ssum == 0.0, jnp.float32(1.0), ssum)
    w1 = real1 * v1 / denom
    w2 = real2 * v2 / denom
    logits_ref[...] = logits
    wb0_ref[...] = jnp.broadcast_to(w1, (S, EP))
    wb1_ref[...] = jnp.broadcast_to(w2, (S, EP))

    colw = lax.broadcasted_iota(jnp.int32, (S, 128), 1)
    oh1 = ((colw == i1) & (colw < E)).astype(jnp.float32)
    oh2 = ((colw == i2) & (colw < E)).astype(jnp.float32)
    cnt = oh1 + oh2
    ri = lax.broadcasted_iota(jnp.int32, (128, 128), 0)
    ci = lax.broadcasted_iota(jnp.int32, (128, 128), 1)
    lstrict = (ci < ri).astype(jnp.float32)
    ustrict = (ri < ci).astype(jnp.float32)
    nch = S // 128
    within = []
    tots = []
    for c in range(nch):
        seg = cnt[c * 128:(c + 1) * 128, :]
        within.append(lax.dot_general(lstrict, seg, (((1,), (0,)), ((), ())),
                                      preferred_element_type=jnp.float32))
        tots.append(jnp.sum(seg, axis=0, keepdims=True))
    within = jnp.concatenate(within, axis=0)
    tots = jnp.concatenate(tots, axis=0)
    ri16 = lax.broadcasted_iota(jnp.int32, (nch, nch), 0)
    ci16 = lax.broadcasted_iota(jnp.int32, (nch, nch), 1)
    l16 = (ci16 < ri16).astype(jnp.float32)
    pref = lax.dot_general(l16, tots, (((1,), (0,)), ((), ())),
                           preferred_element_type=jnp.float32)
    pref_full = jnp.broadcast_to(pref[:, None, :], (nch, 128, 128)).reshape(S, 128)
    rank = within + pref_full
    counts = jnp.sum(tots, axis=0, keepdims=True)
    ci32 = counts.astype(jnp.int32)
    pad = ((ci32 + (T - 1)) // T) * T
    starts = lax.dot_general(pad.astype(jnp.float32), ustrict,
                             (((1,), (0,)), ((), ())),
                             preferred_element_type=jnp.float32)
    pos = starts + rank
    sel1 = jnp.sum(oh1 * pos, axis=1, keepdims=True)
    sel2 = jnp.sum(oh2 * pos, axis=1, keepdims=True)
    dst0_ref[...] = jnp.where(real1 > 0, sel1, jnp.float32(TRASH)).astype(jnp.int32)
    dst1_ref[...] = jnp.where(real2 > 0, sel2, jnp.float32(TRASH)).astype(jnp.int32)
    cnts_ref[...] = jnp.broadcast_to(counts, (8, 128))


_router = pl.pallas_call(
    _router_body,
    out_shape=(
        jax.ShapeDtypeStruct((S, EP), jnp.float32),
        jax.ShapeDtypeStruct((S, 1), jnp.int32),
        jax.ShapeDtypeStruct((S, 1), jnp.int32),
        jax.ShapeDtypeStruct((S, EP), jnp.float32),
        jax.ShapeDtypeStruct((S, EP), jnp.float32),
        jax.ShapeDtypeStruct((8, 128), jnp.float32),
    ),
)

_sc_cache = {}


def _get_dispatch():
    if "dispatch" in _sc_cache:
        return _sc_cache["dispatch"]
    mesh = plsc.VectorSubcoreMesh(
        core_axis_name="c", subcore_axis_name="s", num_cores=NC, num_subcores=NS)

    @functools.partial(
        pl.kernel,
        out_type=jax.ShapeDtypeStruct((NROWS, H), jnp.float32),
        mesh=mesh,
        scratch_types=[
            pltpu.VMEM((2, TPW // 2), jnp.int32),
            pltpu.VMEM((2, TPW // 2), jnp.int32),
            pltpu.VMEM((2, TPW // 2, H), jnp.float32),
            pltpu.SemaphoreType.DMA,
            pltpu.SemaphoreType.DMA,
            pltpu.SemaphoreType.DMA,
            pltpu.SemaphoreType.DMA,
        ],
    )
    def _dispatch(x_hbm, dst0_hbm, dst1_hbm, xg_hbm, idx0_v, idx1_v, rows_v,
                  si, sr0, sr1, ss):
        wid = lax.axis_index("s") * NC + lax.axis_index("c")
        base = wid * TPW
        half = TPW // 2
        hi0 = pltpu.async_copy(dst0_hbm.at[pl.ds(base, half)], idx0_v.at[0], si)
        hi1 = pltpu.async_copy(dst0_hbm.at[pl.ds(base + half, half)],
                               idx0_v.at[1], si)
        hi2 = pltpu.async_copy(dst1_hbm.at[pl.ds(base, half)], idx1_v.at[0], si)
        hi3 = pltpu.async_copy(dst1_hbm.at[pl.ds(base + half, half)],
                               idx1_v.at[1], si)
        hr0 = pltpu.async_copy(x_hbm.at[pl.ds(base, half)], rows_v.at[0], sr0)
        hr1 = pltpu.async_copy(x_hbm.at[pl.ds(base + half, half)], rows_v.at[1],
                               sr1)
        hi0.wait()
        hi1.wait()
        hi2.wait()
        hi3.wait()
        hr0.wait()
        s00 = pltpu.async_copy(rows_v.at[0], xg_hbm.at[idx0_v.at[0]], ss)
        s01 = pltpu.async_copy(rows_v.at[0], xg_hbm.at[idx1_v.at[0]], ss)
        hr1.wait()
        s10 = pltpu.async_copy(rows_v.at[1], xg_hbm.at[idx0_v.at[1]], ss)
        s11 = pltpu.async_copy(rows_v.at[1], xg_hbm.at[idx1_v.at[1]], ss)
        s00.wait()
        s01.wait()
        s10.wait()
        s11.wait()

    _sc_cache["dispatch"] = _dispatch
    return _dispatch


def _ffn_body(be_ref, bmi_ref, bmo_ref, na_ref, xg_ref, w1_ref, w3_ref, w2_ref,
              out_ref):
    b = pl.program_id(0)
    kf = pl.program_id(1)
    active = b < na_ref[0]

    @pl.when(active)
    def _():
        x = xg_ref[...]
        w1c = w1_ref[0]
        w3c = w3_ref[0]
        w2c = w2_ref[0]
        a = lax.dot_general(x, w1c, (((1,), (1,)), ((), ())),
                            preferred_element_type=jnp.float32)
        bb = lax.dot_general(x, w3c, (((1,), (1,)), ((), ())),
                             preferred_element_type=jnp.float32)
        hh = (a * jax.nn.sigmoid(a)) * bb
        part = lax.dot_general(hh, w2c, (((1,), (1,)), ((), ())),
                               preferred_element_type=jnp.float32)

        @pl.when(kf == 0)
        def _():
            out_ref[...] = part

        @pl.when(kf > 0)
        def _():
            out_ref[...] = out_ref[...] + part

    @pl.when(jnp.logical_not(active))
    def _():
        out_ref[...] = jnp.zeros((T, H), jnp.float32)


def _kf_eff(b, kf, na):
    return jnp.where(b < na[0], kf, KFN - 1)


_ffn = pl.pallas_call(
    _ffn_body,
    grid_spec=pltpu.PrefetchScalarGridSpec(
        num_scalar_prefetch=4,
        grid=(G, KFN),
        in_specs=[
            pl.BlockSpec((T, H), lambda b, kf, be, bmi, bmo, na: (bmi[b], 0)),
            pl.BlockSpec((1, FC, H),
                         lambda b, kf, be, bmi, bmo, na:
                         (be[b], _kf_eff(b, kf, na), 0)),
            pl.BlockSpec((1, FC, H),
                         lambda b, kf, be, bmi, bmo, na:
                         (be[b], _kf_eff(b, kf, na), 0)),
            pl.BlockSpec((1, H, FC),
                         lambda b, kf, be, bmi, bmo, na:
                         (be[b], 0, _kf_eff(b, kf, na))),
        ],
        out_specs=pl.BlockSpec((T, H), lambda b, kf, be, bmi, bmo, na: (bmo[b], 0)),
    ),
    out_shape=jax.ShapeDtypeStruct((NROWS, H), jnp.float32),
)


def _get_combine():
    if "combine" in _sc_cache:
        return _sc_cache["combine"]
    mesh = plsc.VectorSubcoreMesh(
        core_axis_name="c", subcore_axis_name="s", num_cores=NC, num_subcores=NS)

    nchk = TPW // CCH

    @functools.partial(
        pl.kernel,
        out_type=jax.ShapeDtypeStruct((S, H), jnp.float32),
        mesh=mesh,
        scratch_types=[
            pltpu.VMEM((2, CCH), jnp.int32),
            pltpu.VMEM((2, CCH), jnp.int32),
            pltpu.VMEM((2, CCH, EP), jnp.float32),
            pltpu.VMEM((2, CCH, EP), jnp.float32),
            pltpu.VMEM((2, CCH, H), jnp.float32),
            pltpu.VMEM((2, CCH, H), jnp.float32),
            pltpu.VMEM((2, CCH, H), jnp.float32),
            pltpu.SemaphoreType.DMA,
            pltpu.SemaphoreType.DMA,
            pltpu.SemaphoreType.DMA,
            pltpu.SemaphoreType.DMA,
        ],
    )
    def _combine(y_hbm, dst0_hbm, dst1_hbm, wb0_hbm, wb1_hbm, out_hbm,
                 idx0_v, idx1_v, w0_v, w1_v, r0_v, r1_v, o_v,
                 sg0, sg1, so0, so1):
        wid = lax.axis_index("s") * NC + lax.axis_index("c")
        base = wid * TPW
        sg = (sg0, sg1)
        so = (so0, so1)

        def load(ch, p):
            tb = base + ch * CCH
            pltpu.sync_copy(dst0_hbm.at[pl.ds(tb, CCH)], idx0_v.at[p])
            pltpu.sync_copy(dst1_hbm.at[pl.ds(tb, CCH)], idx1_v.at[p])
            pltpu.sync_copy(wb0_hbm.at[pl.ds(tb, CCH), :], w0_v.at[p])
            pltpu.sync_copy(wb1_hbm.at[pl.ds(tb, CCH), :], w1_v.at[p])
            return (pltpu.async_copy(y_hbm.at[idx0_v.at[p]], r0_v.at[p], sg[p]),
                    pltpu.async_copy(y_hbm.at[idx1_v.at[p]], r1_v.at[p], sg[p]))

        pending = load(0, 0)
        wb = [None, None]
        for ch in range(nchk):
            p = ch & 1
            nxt = load(ch + 1, (ch + 1) & 1) if ch + 1 < nchk else None
            pending[0].wait()
            pending[1].wait()
            if wb[p] is not None:
                wb[p].wait()

            def tok(i, carry):
                w0 = w0_v[p, i, :]
                w1 = w1_v[p, i, :]
                for j in range(H // 16):
                    sl = pl.ds(j * 16, 16)
                    o_v[p, i, sl] = w0 * r0_v[p, i, sl] + w1 * r1_v[p, i, sl]
                return carry

            lax.fori_loop(0, CCH, tok, 0)
            tb = base + ch * CCH
            wb[p] = pltpu.async_copy(o_v.at[p], out_hbm.at[pl.ds(tb, CCH), :],
                                     so[p])
            pending = nxt
        for h in wb:
            if h is not None:
                h.wait()

    _sc_cache["combine"] = _combine
    return _combine


def kernel(hidden_states, gate_w, gate2_w, W1, W2, W3):
    b, s, h = hidden_states.shape
    x = hidden_states.reshape(s, h)
    wg = jnp.pad(jnp.concatenate([gate_w, gate2_w], axis=0),
                 ((0, EP - NE), (0, 0)))
    logits, dst0, dst1, wb0, wb1, cnts = _router(x, wg)
    dst0 = dst0.reshape(s)
    dst1 = dst1.reshape(s)
    c8 = cnts[0, :E].astype(jnp.int32)
    pad8 = ((c8 + T - 1) // T) * T
    ends = jnp.cumsum(pad8)
    na = ends[E - 1] // T
    bidx = jnp.arange(G, dtype=jnp.int32)
    be_full = jnp.minimum(
        jnp.sum((bidx[:, None] * T >= ends[None, :]).astype(jnp.int32), axis=1),
        E - 1)
    bmi = jnp.minimum(bidx, jnp.maximum(na, 1) - 1).astype(jnp.int32)
    bmo = jnp.where(bidx < na, bidx, G - 1).astype(jnp.int32)
    be = be_full[bmi].astype(jnp.int32)
    na_arr = jnp.reshape(na, (1,)).astype(jnp.int32)

    xg = _get_dispatch()(x, dst0, dst1)
    y = _ffn(be, bmi, bmo, na_arr, xg, W1, W3, W2)
    out = _get_combine()(y, dst0, dst1, wb0, wb1)
    return out.reshape(b, s, h), logits[:, :NE]


# speedup vs baseline: 1.2352x; 1.2352x over previous
import functools

import jax
import jax.numpy as jnp
from jax import lax
from jax.experimental import pallas as pl
from jax.experimental.pallas import tpu as pltpu
from jax.experimental.pallas import tpu_sc as plsc

E = 8
NE = 10
EP = 16
S = 2048
H = 1024
FF = 4096
T = 512
G = 16
FC = 1024
KFN = FF // FC
NROWS = G * T
TRASH = NROWS - 1
NC, NS = 2, 16
NW = NC * NS
TPW = S // NW
CCH = 16


def _router_body(x_ref, wg_ref, logits_ref, dst0_ref, dst1_ref,
                 wb0_ref, wb1_ref, cnts_ref):
    x = x_ref[...]
    wg = wg_ref[...]
    logits = lax.dot_general(x, wg, (((1,), (1,)), ((), ())),
                             preferred_element_type=jnp.float32)
    col = lax.broadcasted_iota(jnp.int32, (S, EP), 1)
    lm = jnp.where(col < NE, logits, jnp.float32(-1e30))
    m = jnp.max(lm, axis=1, keepdims=True)
    p = jnp.exp(lm - m)
    probs = p / jnp.sum(p, axis=1, keepdims=True)
    v1 = jnp.max(probs, axis=1, keepdims=True)
    i1 = jnp.min(jnp.where(probs == v1, col, EP + 1), axis=1, keepdims=True)
    probs2 = jnp.where(col == i1, jnp.float32(-1.0), probs)
    v2 = jnp.max(probs2, axis=1, keepdims=True)
    i2 = jnp.min(jnp.where(probs2 == v2, col, EP + 1), axis=1, keepdims=True)
    real1 = (i1 < E).astype(jnp.float32)
    real2 = (i2 < E).astype(jnp.float32)
    ssum = v1 * real1 + v2 * real2
    denom = jnp.where(ssum == 0.0, jnp.float32(1.0), ssum)
    w1 = real1 * v1 / denom
    w2 = real2 * v2 / denom
    logits_ref[...] = logits
    wb0_ref[...] = jnp.broadcast_to(w1, (S, EP))
    wb1_ref[...] = jnp.broadcast_to(w2, (S, EP))

    colw = lax.broadcasted_iota(jnp.int32, (S, 128), 1)
    oh1 = ((colw == i1) & (colw < E)).astype(jnp.float32)
    oh2 = ((colw == i2) & (colw < E)).astype(jnp.float32)
    cnt = oh1 + oh2
    ri = lax.broadcasted_iota(jnp.int32, (128, 128), 0)
    ci = lax.broadcasted_iota(jnp.int32, (128, 128), 1)
    lstrict = (ci < ri).astype(jnp.float32)
    ustrict = (ri < ci).astype(jnp.float32)
    nch = S // 128
    within = []
    tots = []
    for c in range(nch):
        seg = cnt[c * 128:(c + 1) * 128, :]
        within.append(lax.dot_general(lstrict, seg, (((1,), (0,)), ((), ())),
                                      preferred_element_type=jnp.float32))
        tots.append(jnp.sum(seg, axis=0, keepdims=True))
    within = jnp.concatenate(within, axis=0)
    tots = jnp.concatenate(tots, axis=0)
    ri16 = lax.broadcasted_iota(jnp.int32, (nch, nch), 0)
    ci16 = lax.broadcasted_iota(jnp.int32, (nch, nch), 1)
    l16 = (ci16 < ri16).astype(jnp.float32)
    pref = lax.dot_general(l16, tots, (((1,), (0,)), ((), ())),
                           preferred_element_type=jnp.float32)
    pref_full = jnp.broadcast_to(pref[:, None, :], (nch, 128, 128)).reshape(S, 128)
    rank = within + pref_full
    counts = jnp.sum(tots, axis=0, keepdims=True)
    ci32 = counts.astype(jnp.int32)
    pad = ((ci32 + (T - 1)) // T) * T
    starts = lax.dot_general(pad.astype(jnp.float32), ustrict,
                             (((1,), (0,)), ((), ())),
                             preferred_element_type=jnp.float32)
    pos = starts + rank
    sel1 = jnp.sum(oh1 * pos, axis=1, keepdims=True)
    sel2 = jnp.sum(oh2 * pos, axis=1, keepdims=True)
    dst0_ref[...] = jnp.where(real1 > 0, sel1, jnp.float32(TRASH)).astype(jnp.int32)
    dst1_ref[...] = jnp.where(real2 > 0, sel2, jnp.float32(TRASH)).astype(jnp.int32)
    cnts_ref[...] = jnp.broadcast_to(counts, (8, 128))


_router = pl.pallas_call(
    _router_body,
    out_shape=(
        jax.ShapeDtypeStruct((S, EP), jnp.float32),
        jax.ShapeDtypeStruct((S, 1), jnp.int32),
        jax.ShapeDtypeStruct((S, 1), jnp.int32),
        jax.ShapeDtypeStruct((S, EP), jnp.float32),
        jax.ShapeDtypeStruct((S, EP), jnp.float32),
        jax.ShapeDtypeStruct((8, 128), jnp.float32),
    ),
)

_sc_cache = {}


def _get_dispatch():
    if "dispatch" in _sc_cache:
        return _sc_cache["dispatch"]
    mesh = plsc.VectorSubcoreMesh(
        core_axis_name="c", subcore_axis_name="s", num_cores=NC, num_subcores=NS)

    @functools.partial(
        pl.kernel,
        out_type=jax.ShapeDtypeStruct((NROWS, H), jnp.float32),
        mesh=mesh,
        scratch_types=[
            pltpu.VMEM((2, TPW // 2), jnp.int32),
            pltpu.VMEM((2, TPW // 2), jnp.int32),
            pltpu.VMEM((2, TPW // 2, H), jnp.float32),
            pltpu.SemaphoreType.DMA,
            pltpu.SemaphoreType.DMA,
            pltpu.SemaphoreType.DMA,
            pltpu.SemaphoreType.DMA,
        ],
    )
    def _dispatch(x_hbm, dst0_hbm, dst1_hbm, xg_hbm, idx0_v, idx1_v, rows_v,
                  si, sr0, sr1, ss):
        wid = lax.axis_index("s") * NC + lax.axis_index("c")
        base = wid * TPW
        half = TPW // 2
        hi0 = pltpu.async_copy(dst0_hbm.at[pl.ds(base, half)], idx0_v.at[0], si)
        hi1 = pltpu.async_copy(dst0_hbm.at[pl.ds(base + half, half)],
                               idx0_v.at[1], si)
        hi2 = pltpu.async_copy(dst1_hbm.at[pl.ds(base, half)], idx1_v.at[0], si)
        hi3 = pltpu.async_copy(dst1_hbm.at[pl.ds(base + half, half)],
                               idx1_v.at[1], si)
        hr0 = pltpu.async_copy(x_hbm.at[pl.ds(base, half)], rows_v.at[0], sr0)
        hr1 = pltpu.async_copy(x_hbm.at[pl.ds(base + half, half)], rows_v.at[1],
                               sr1)
        hi0.wait()
        hi1.wait()
        hi2.wait()
        hi3.wait()
        hr0.wait()
        s00 = pltpu.async_copy(rows_v.at[0], xg_hbm.at[idx0_v.at[0]], ss)
        s01 = pltpu.async_copy(rows_v.at[0], xg_hbm.at[idx1_v.at[0]], ss)
        hr1.wait()
        s10 = pltpu.async_copy(rows_v.at[1], xg_hbm.at[idx0_v.at[1]], ss)
        s11 = pltpu.async_copy(rows_v.at[1], xg_hbm.at[idx1_v.at[1]], ss)
        s00.wait()
        s01.wait()
        s10.wait()
        s11.wait()

    _sc_cache["dispatch"] = _dispatch
    return _dispatch


def _ffn_body(be_ref, bmi_ref, bmo_ref, na_ref, xg_ref, w1_ref, w3_ref, w2_ref,
              out_ref):
    b = pl.program_id(0)
    kf = pl.program_id(1)
    active = b < na_ref[0]

    @pl.when(active)
    def _():
        x = xg_ref[...]
        w1c = w1_ref[0]
        w3c = w3_ref[0]
        w2c = w2_ref[0]
        a = lax.dot_general(x, w1c, (((1,), (1,)), ((), ())),
                            preferred_element_type=jnp.float32)
        bb = lax.dot_general(x, w3c, (((1,), (1,)), ((), ())),
                             preferred_element_type=jnp.float32)
        hh = (a * jax.nn.sigmoid(a)) * bb
        part = lax.dot_general(hh, w2c, (((1,), (1,)), ((), ())),
                               preferred_element_type=jnp.float32)

        @pl.when(kf == 0)
        def _():
            out_ref[...] = part

        @pl.when(kf > 0)
        def _():
            out_ref[...] = out_ref[...] + part

    @pl.when(jnp.logical_and(jnp.logical_not(active),
                             jnp.logical_and(b == na_ref[0], kf == 0)))
    def _():
        out_ref[...] = jnp.zeros((T, H), jnp.float32)


def _kf_eff(b, kf, na):
    return jnp.where(b < na[0], kf, KFN - 1)


_ffn = pl.pallas_call(
    _ffn_body,
    grid_spec=pltpu.PrefetchScalarGridSpec(
        num_scalar_prefetch=4,
        grid=(G, KFN),
        in_specs=[
            pl.BlockSpec((T, H), lambda b, kf, be, bmi, bmo, na: (bmi[b], 0)),
            pl.BlockSpec((1, FC, H),
                         lambda b, kf, be, bmi, bmo, na:
                         (be[b], _kf_eff(b, kf, na), 0)),
            pl.BlockSpec((1, FC, H),
                         lambda b, kf, be, bmi, bmo, na:
                         (be[b], _kf_eff(b, kf, na), 0)),
            pl.BlockSpec((1, H, FC),
                         lambda b, kf, be, bmi, bmo, na:
                         (be[b], 0, _kf_eff(b, kf, na))),
        ],
        out_specs=pl.BlockSpec((T, H), lambda b, kf, be, bmi, bmo, na: (bmo[b], 0)),
    ),
    out_shape=jax.ShapeDtypeStruct((NROWS, H), jnp.float32),
)


def _get_combine():
    if "combine" in _sc_cache:
        return _sc_cache["combine"]
    mesh = plsc.VectorSubcoreMesh(
        core_axis_name="c", subcore_axis_name="s", num_cores=NC, num_subcores=NS)

    nchk = TPW // CCH

    @functools.partial(
        pl.kernel,
        out_type=jax.ShapeDtypeStruct((S, H), jnp.float32),
        mesh=mesh,
        scratch_types=[
            pltpu.VMEM((2, CCH), jnp.int32),
            pltpu.VMEM((2, CCH), jnp.int32),
            pltpu.VMEM((2, CCH, EP), jnp.float32),
            pltpu.VMEM((2, CCH, EP), jnp.float32),
            pltpu.VMEM((2, CCH, H), jnp.float32),
            pltpu.VMEM((2, CCH, H), jnp.float32),
            pltpu.VMEM((2, CCH, H), jnp.float32),
            pltpu.SemaphoreType.DMA,
            pltpu.SemaphoreType.DMA,
            pltpu.SemaphoreType.DMA,
            pltpu.SemaphoreType.DMA,
        ],
    )
    def _combine(y_hbm, dst0_hbm, dst1_hbm, wb0_hbm, wb1_hbm, out_hbm,
                 idx0_v, idx1_v, w0_v, w1_v, r0_v, r1_v, o_v,
                 sg0, sg1, so0, so1):
        wid = lax.axis_index("s") * NC + lax.axis_index("c")
        base = wid * TPW
        sg = (sg0, sg1)
        so = (so0, so1)

        def load(ch, p):
            tb = base + ch * CCH
            pltpu.sync_copy(dst0_hbm.at[pl.ds(tb, CCH)], idx0_v.at[p])
            pltpu.sync_copy(dst1_hbm.at[pl.ds(tb, CCH)], idx1_v.at[p])
            pltpu.sync_copy(wb0_hbm.at[pl.ds(tb, CCH), :], w0_v.at[p])
            pltpu.sync_copy(wb1_hbm.at[pl.ds(tb, CCH), :], w1_v.at[p])
            return (pltpu.async_copy(y_hbm.at[idx0_v.at[p]], r0_v.at[p], sg[p]),
                    pltpu.async_copy(y_hbm.at[idx1_v.at[p]], r1_v.at[p], sg[p]))

        pending = load(0, 0)
        wb = [None, None]
        for ch in range(nchk):
            p = ch & 1
            nxt = load(ch + 1, (ch + 1) & 1) if ch + 1 < nchk else None
            pending[0].wait()
            pending[1].wait()
            if wb[p] is not None:
                wb[p].wait()

            def tok(i, carry):
                w0 = w0_v[p, i, :]
                w1 = w1_v[p, i, :]
                for j in range(H // 16):
                    sl = pl.ds(j * 16, 16)
                    o_v[p, i, sl] = w0 * r0_v[p, i, sl] + w1 * r1_v[p, i, sl]
                return carry

            lax.fori_loop(0, CCH, tok, 0)
            tb = base + ch * CCH
            wb[p] = pltpu.async_copy(o_v.at[p], out_hbm.at[pl.ds(tb, CCH), :],
                                     so[p])
            pending = nxt
        for h in wb:
            if h is not None:
                h.wait()

    _sc_cache["combine"] = _combine
    return _combine


def kernel(hidden_states, gate_w, gate2_w, W1, W2, W3):
    b, s, h = hidden_states.shape
    x = hidden_states.reshape(s, h)
    wg = jnp.pad(jnp.concatenate([gate_w, gate2_w], axis=0),
                 ((0, EP - NE), (0, 0)))
    logits, dst0, dst1, wb0, wb1, cnts = _router(x, wg)
    dst0 = dst0.reshape(s)
    dst1 = dst1.reshape(s)
    c8 = cnts[0, :E].astype(jnp.int32)
    pad8 = ((c8 + T - 1) // T) * T
    ends = jnp.cumsum(pad8)
    na = ends[E - 1] // T
    bidx = jnp.arange(G, dtype=jnp.int32)
    be_full = jnp.minimum(
        jnp.sum((bidx[:, None] * T >= ends[None, :]).astype(jnp.int32), axis=1),
        E - 1)
    bmi = jnp.minimum(bidx, jnp.maximum(na, 1) - 1).astype(jnp.int32)
    bmo = jnp.where(bidx < na, bidx, G - 1).astype(jnp.int32)
    be = be_full[bmi].astype(jnp.int32)
    na_arr = jnp.reshape(na, (1,)).astype(jnp.int32)

    xg = _get_dispatch()(x, dst0, dst1)
    y = _ffn(be, bmi, bmo, na_arr, xg, W1, W3, W2)
    out = _get_combine()(y, dst0, dst1, wb0, wb1)
    return out.reshape(b, s, h), logits[:, :NE]
